# Initial kernel scaffold; baseline (speedup 1.0000x reference)
import jax
import jax.numpy as jnp
from jax.experimental import pallas as pl


def kernel(voxel_features, voxel_coords, batch_size):
    raise NotImplementedError("write your pallas kernel here")


# speedup vs baseline: 3.4605x; 3.4605x over previous
import functools

import jax
import jax.numpy as jnp
from jax import lax
from jax.experimental import pallas as pl
from jax.experimental.pallas import tpu as pltpu
from jax.experimental.pallas import tpu_sc as plsc

NB = 4
NW = 32
SCAT = 128
SUB = 2048
NSUB = 7
CHUNK = SUB * NSUB
NPAD = CHUNK * NW
SLABS = NPAD // 128
CSLABS = SLABS // NW
N = 400000
PART = N % SUB

_MESH = plsc.VectorSubcoreMesh(core_axis_name="c", subcore_axis_name="s")
_CP = pltpu.CompilerParams(needs_layout_passes=False, use_tc_tiling_on_sc=False)


def _prep_body(c_ref, f_ref, planes_ref, b_ref, cnt_ref):
    x = c_ref[:]
    f = f_ref[:]
    pc = jnp.concatenate([
        x[:, 0:1, :],
        (x[:, 3:4, :] + 0.5) * 0.05,
        (x[:, 2:3, :] + 0.5) * 0.05 - 40.0,
        (x[:, 1:2, :] + 0.5) * 0.1 - 3.0,
    ], axis=1)
    combined = jnp.concatenate([f, pc], axis=1)
    planes_ref[:] = combined.transpose(1, 0, 2)
    bi = x[:, 0, :].astype(jnp.int32)
    b_ref[:] = bi
    cl = lax.broadcasted_iota(jnp.int32, (1, 128), 1)
    cv = jnp.zeros((1, 128), jnp.int32)
    for b in range(NB):
        cv = jnp.where(cl == b, jnp.sum((bi == b).astype(jnp.int32)), cv)
    cnt_ref[:] = cv.reshape(1, 1, 128)


def _tc_prep(coords_slab, feats_slab):
    return pl.pallas_call(
        _prep_body,
        grid=(NW,),
        in_specs=[
            pl.BlockSpec((CSLABS, 4, 128), lambda i: (i, 0, 0)),
            pl.BlockSpec((CSLABS, 4, 128), lambda i: (i, 0, 0)),
        ],
        out_specs=[
            pl.BlockSpec((8, CSLABS, 128), lambda i: (0, i, 0)),
            pl.BlockSpec((CSLABS, 128), lambda i: (i, 0)),
            pl.BlockSpec((1, 1, 128), lambda i: (i, 0, 0)),
        ],
        out_shape=[
            jax.ShapeDtypeStruct((8, SLABS, 128), jnp.float32),
            jax.ShapeDtypeStruct((SLABS, 128), jnp.int32),
            jax.ShapeDtypeStruct((NW, 1, 128), jnp.int32),
        ],
    )(coords_slab, feats_slab)


@functools.partial(
    pl.kernel,
    mesh=_MESH,
    compiler_params=_CP,
    out_type=jax.ShapeDtypeStruct((NPAD, 8), jnp.float32),
    scratch_types=[
        pltpu.VMEM((NW, 1, 128), jnp.int32),
        pltpu.VMEM((SUB,), jnp.int32),
        pltpu.VMEM((8, SUB), jnp.float32),
        pltpu.VMEM((SUB, 8), jnp.float32),
        pltpu.VMEM((SUB // SCAT, SCAT), jnp.int32),
        pltpu.SemaphoreType.DMA,
    ],
)
def _sc_scatter(planes_hbm, bidx_hbm, cnt_hbm, out_hbm,
                cntbuf, bbuf, pjbuf, pbuf, destbuf, sem):
    wid = lax.axis_index("s") * 2 + lax.axis_index("c")
    base = wid * CHUNK
    lanes = lax.iota(jnp.int32, 16)
    zeros = jnp.zeros((16,), jnp.int32)

    pltpu.sync_copy(cnt_hbm, cntbuf)
    wid_v = jnp.full((16,), wid, jnp.int32)
    totv = zeros
    minev = zeros
    for w in range(NW):
        c_w = cntbuf[w, 0, pl.ds(0, 16)]
        totv = totv + c_w
        minev = minev + jnp.where(jnp.full((16,), w, jnp.int32) < wid_v,
                                  c_w, zeros)
    startsv = plsc.cumsum(totv) - totv + minev
    runs0 = tuple(
        jnp.full((16,), jnp.sum(jnp.where(lanes == b, startsv, zeros)),
                 jnp.int32)
        for b in range(NB))

    def sub_body(j, runs):
        row0 = base + j * SUB
        pltpu.sync_copy(bidx_hbm.at[pl.ds(row0, SUB)], bbuf)
        for p in range(8):
            pltpu.sync_copy(planes_hbm.at[p, pl.ds(row0, SUB)],
                            pjbuf.at[p])
        handles = []
        for jj in range(SUB // SCAT):
            def dg(g, runs):
                off = jj * SCAT + g * 16
                rows = off + lanes
                for p in range(8):
                    v = pjbuf[p, pl.ds(off, 16)]
                    plsc.store_scatter(pbuf, [rows, zeros + p], v)
                bvec = bbuf[pl.ds(off, 16)]
                dest = zeros
                new_runs = []
                for b in range(NB):
                    m = bvec == b
                    mi = m.astype(jnp.int32)
                    excl = plsc.cumsum(mi) - mi
                    dest = dest + jnp.where(m, runs[b] + excl, 0)
                    new_runs.append(
                        runs[b] + plsc.all_reduce_population_count(m))
                dest = jnp.where(bvec >= NB, row0 + rows, dest)
                destbuf[jj, pl.ds(g * 16, 16)] = dest
                return tuple(new_runs)

            runs = lax.fori_loop(0, SCAT // 16, dg, runs)
            handles.append(pltpu.async_copy(
                pbuf.at[pl.ds(jj * SCAT, SCAT)],
                out_hbm.at[destbuf.at[jj]], sem))
        for h in handles:
            h.wait()
        return runs

    lax.fori_loop(0, NSUB, sub_body, runs0)


@functools.partial(
    pl.kernel,
    mesh=_MESH,
    compiler_params=_CP,
    out_type=[
        jax.ShapeDtypeStruct((4, N), jnp.float32),
        jax.ShapeDtypeStruct((4, N), jnp.float32),
    ],
    scratch_types=[
        pltpu.VMEM((SUB, 8), jnp.float32),
        pltpu.VMEM((8, SUB), jnp.float32),
    ],
)
def _sc_unpack(packed_hbm, outf_hbm, outpc_hbm, qbuf, pjbuf):
    wid = lax.axis_index("s") * 2 + lax.axis_index("c")
    base = wid * CHUNK
    lanes = lax.iota(jnp.int32, 16)
    zeros = jnp.zeros((16,), jnp.int32)

    def sub_body(j, carry):
        row0 = base + j * SUB

        @pl.when(row0 < N)
        def _():
            pltpu.sync_copy(packed_hbm.at[pl.ds(row0, SUB)], qbuf)

            def grp(g, carry):
                off = g * 16
                rows = off + lanes
                for p in range(8):
                    w = plsc.load_gather(qbuf, [rows, zeros + p])
                    pjbuf[p, pl.ds(off, 16)] = w
                return carry

            lax.fori_loop(0, SUB // 16, grp, 0)

            @pl.when(row0 + SUB <= N)
            def _():
                for p in range(4):
                    pltpu.sync_copy(pjbuf.at[p],
                                    outf_hbm.at[p, pl.ds(row0, SUB)])
                    pltpu.sync_copy(pjbuf.at[p + 4],
                                    outpc_hbm.at[p, pl.ds(row0, SUB)])

            @pl.when(row0 + SUB > N)
            def _():
                for p in range(4):
                    pltpu.sync_copy(pjbuf.at[p, pl.ds(0, PART)],
                                    outf_hbm.at[p, pl.ds(row0, PART)])
                    pltpu.sync_copy(pjbuf.at[p + 4, pl.ds(0, PART)],
                                    outpc_hbm.at[p, pl.ds(row0, PART)])

        return carry

    lax.fori_loop(0, NSUB, sub_body, 0)


def _slab(a):
    return jnp.swapaxes(jnp.swapaxes(a, 0, 1).reshape(4, N // 128, 128), 0, 1)


def kernel(voxel_features, voxel_coords, batch_size):
    padslab = SLABS - N // 128
    cs = jnp.concatenate(
        [_slab(voxel_coords),
         jnp.full((padslab, 4, 128), float(NB), jnp.float32)], axis=0)
    fs = jnp.concatenate(
        [_slab(voxel_features),
         jnp.zeros((padslab, 4, 128), jnp.float32)], axis=0)
    planes3d, bidx2d, cnts = _tc_prep(cs, fs)
    packed_sorted = _sc_scatter(planes3d.reshape(8, NPAD),
                                bidx2d.reshape(NPAD), cnts)
    outf2d, outpc2d = _sc_unpack(packed_sorted)
    return jnp.swapaxes(outf2d, 0, 1), jnp.swapaxes(outpc2d, 0, 1)


# speedup vs baseline: 4.5879x; 1.3258x over previous
import functools

import jax
import jax.numpy as jnp
from jax import lax
from jax.experimental import pallas as pl
from jax.experimental.pallas import tpu as pltpu
from jax.experimental.pallas import tpu_sc as plsc

NB = 4
NW = 32
NCH = 28
SCAT = 128
SUB = 2048
NSUB = 7
CHUNK = SUB * NSUB
NPAD = CHUNK * NCH
SLABS = NPAD // 128
CSLABS = SLABS // NCH
N = 400000
PART = N % SUB

_MESH = plsc.VectorSubcoreMesh(core_axis_name="c", subcore_axis_name="s")
_CP = pltpu.CompilerParams(needs_layout_passes=False, use_tc_tiling_on_sc=False)


def _prep_body(c_ref, f_ref, planes_ref, b_ref, cnt_ref):
    i = pl.program_id(0)
    x = c_ref[:]
    f = f_ref[:]
    pc = jnp.concatenate([
        x[:, 0:1, :],
        (x[:, 3:4, :] + 0.5) * 0.05,
        (x[:, 2:3, :] + 0.5) * 0.05 - 40.0,
        (x[:, 1:2, :] + 0.5) * 0.1 - 3.0,
    ], axis=1)
    combined = jnp.concatenate([f, pc], axis=1)
    planes_ref[:] = combined.transpose(1, 0, 2)
    s_iota = lax.broadcasted_iota(jnp.int32, (CSLABS, 128), 0)
    l_iota = lax.broadcasted_iota(jnp.int32, (CSLABS, 128), 1)
    grow = 128 * (CSLABS * i + s_iota) + l_iota
    bi = jnp.where(grow < N, x[:, 0, :].astype(jnp.int32), NB)
    b_ref[:] = bi
    cl = lax.broadcasted_iota(jnp.int32, (1, 128), 1)
    cv = jnp.zeros((1, 128), jnp.int32)
    for b in range(NB):
        cv = jnp.where(cl == b, jnp.sum((bi == b).astype(jnp.int32)), cv)
    cnt_ref[:] = cv.reshape(1, 1, 128)


def _tc_prep(coords_slab, feats_slab):
    return pl.pallas_call(
        _prep_body,
        grid=(NCH,),
        in_specs=[
            pl.BlockSpec((CSLABS, 4, 128), lambda i: (i, 0, 0)),
            pl.BlockSpec((CSLABS, 4, 128), lambda i: (i, 0, 0)),
        ],
        out_specs=[
            pl.BlockSpec((8, CSLABS, 128), lambda i: (0, i, 0)),
            pl.BlockSpec((CSLABS, 128), lambda i: (i, 0)),
            pl.BlockSpec((1, 1, 128), lambda i: (i, 0, 0)),
        ],
        out_shape=[
            jax.ShapeDtypeStruct((8, SLABS, 128), jnp.float32),
            jax.ShapeDtypeStruct((SLABS, 128), jnp.int32),
            jax.ShapeDtypeStruct((NCH, 1, 128), jnp.int32),
        ],
    )(coords_slab, feats_slab)


@functools.partial(
    pl.kernel,
    mesh=_MESH,
    compiler_params=_CP,
    out_type=jax.ShapeDtypeStruct((NPAD, 8), jnp.float32),
    scratch_types=[
        pltpu.VMEM((NCH, 1, 128), jnp.int32),
        pltpu.VMEM((SUB,), jnp.int32),
        pltpu.VMEM((8, SUB), jnp.float32),
        pltpu.VMEM((SUB, 8), jnp.float32),
        pltpu.VMEM((SUB // SCAT, SCAT), jnp.int32),
        pltpu.SemaphoreType.DMA,
    ],
)
def _sc_scatter(planes_hbm, bidx_hbm, cnt_hbm, out_hbm,
                cntbuf, bbuf, pjbuf, pbuf, destbuf, sem):
    wid = lax.axis_index("s") * 2 + lax.axis_index("c")
    base = wid * CHUNK
    lanes = lax.iota(jnp.int32, 16)
    zeros = jnp.zeros((16,), jnp.int32)

    pltpu.sync_copy(cnt_hbm, cntbuf)
    wid_v = jnp.full((16,), wid, jnp.int32)
    totv = zeros
    minev = zeros
    for w in range(NCH):
        c_w = cntbuf[w, 0, pl.ds(0, 16)]
        totv = totv + c_w
        minev = minev + jnp.where(jnp.full((16,), w, jnp.int32) < wid_v,
                                  c_w, zeros)
    startsv = plsc.cumsum(totv) - totv + minev
    runs0 = tuple(
        jnp.full((16,), jnp.sum(jnp.where(lanes == b, startsv, zeros)),
                 jnp.int32)
        for b in range(NB))

    def sub_body(j, runs):
        row0 = base + j * SUB
        h1 = pltpu.async_copy(bidx_hbm.at[pl.ds(row0, SUB)], bbuf, sem)
        h2 = pltpu.async_copy(planes_hbm.at[:, pl.ds(row0, SUB)], pjbuf, sem)
        h1.wait()
        h2.wait()
        handles = []
        for jj in range(SUB // SCAT):
            def dg(g, runs):
                off = jj * SCAT + g * 16
                rows = off + lanes
                for p in range(8):
                    v = pjbuf[p, pl.ds(off, 16)]
                    plsc.store_scatter(pbuf, [rows, zeros + p], v)
                bvec = bbuf[pl.ds(off, 16)]
                dest = zeros
                new_runs = []
                for b in range(NB):
                    m = bvec == b
                    mi = m.astype(jnp.int32)
                    excl = plsc.cumsum(mi) - mi
                    dest = dest + jnp.where(m, runs[b] + excl, 0)
                    new_runs.append(
                        runs[b] + plsc.all_reduce_population_count(m))
                dest = jnp.where(bvec >= NB, row0 + rows, dest)
                destbuf[jj, pl.ds(g * 16, 16)] = dest
                return tuple(new_runs)

            runs = lax.fori_loop(0, SCAT // 16, dg, runs)
            handles.append(pltpu.async_copy(
                pbuf.at[pl.ds(jj * SCAT, SCAT)],
                out_hbm.at[destbuf.at[jj]], sem))
        for h in handles:
            h.wait()
        return runs

    @pl.when(wid < NCH)
    def _():
        lax.fori_loop(0, NSUB, sub_body, runs0)


@functools.partial(
    pl.kernel,
    mesh=_MESH,
    compiler_params=_CP,
    out_type=[
        jax.ShapeDtypeStruct((4, N), jnp.float32),
        jax.ShapeDtypeStruct((4, N), jnp.float32),
    ],
    scratch_types=[
        pltpu.VMEM((SUB, 8), jnp.float32),
        pltpu.VMEM((8, SUB), jnp.float32),
        pltpu.SemaphoreType.DMA,
    ],
)
def _sc_unpack(packed_hbm, outf_hbm, outpc_hbm, qbuf, pjbuf, sem):
    wid = lax.axis_index("s") * 2 + lax.axis_index("c")
    base = wid * CHUNK
    lanes = lax.iota(jnp.int32, 16)
    zeros = jnp.zeros((16,), jnp.int32)

    def sub_body(j, carry):
        row0 = base + j * SUB

        @pl.when(row0 < N)
        def _():
            pltpu.sync_copy(packed_hbm.at[pl.ds(row0, SUB)], qbuf)

            def grp(g, carry):
                off = g * 16
                rows = off + lanes
                for p in range(8):
                    w = plsc.load_gather(qbuf, [rows, zeros + p])
                    pjbuf[p, pl.ds(off, 16)] = w
                return carry

            lax.fori_loop(0, SUB // 16, grp, 0)

            @pl.when(row0 + SUB <= N)
            def _():
                h1 = pltpu.async_copy(
                    pjbuf.at[pl.ds(0, 4)],
                    outf_hbm.at[:, pl.ds(row0, SUB)], sem)
                h2 = pltpu.async_copy(
                    pjbuf.at[pl.ds(4, 4)],
                    outpc_hbm.at[:, pl.ds(row0, SUB)], sem)
                h1.wait()
                h2.wait()

            @pl.when(row0 + SUB > N)
            def _():
                h1 = pltpu.async_copy(
                    pjbuf.at[pl.ds(0, 4), pl.ds(0, PART)],
                    outf_hbm.at[:, pl.ds(row0, PART)], sem)
                h2 = pltpu.async_copy(
                    pjbuf.at[pl.ds(4, 4), pl.ds(0, PART)],
                    outpc_hbm.at[:, pl.ds(row0, PART)], sem)
                h1.wait()
                h2.wait()

        return carry

    @pl.when(wid < NCH)
    def _():
        lax.fori_loop(0, NSUB, sub_body, 0)


def _slab(a):
    return jnp.swapaxes(jnp.swapaxes(a, 0, 1).reshape(4, N // 128, 128), 0, 1)


def kernel(voxel_features, voxel_coords, batch_size):
    planes3d, bidx2d, cnts = _tc_prep(_slab(voxel_coords),
                                      _slab(voxel_features))
    packed_sorted = _sc_scatter(planes3d.reshape(8, NPAD),
                                bidx2d.reshape(NPAD), cnts)
    outf2d, outpc2d = _sc_unpack(packed_sorted)
    return jnp.swapaxes(outf2d, 0, 1), jnp.swapaxes(outpc2d, 0, 1)


# speedup vs baseline: 5.0611x; 1.1031x over previous
import functools

import jax
import jax.numpy as jnp
from jax import lax
from jax.experimental import pallas as pl
from jax.experimental.pallas import tpu as pltpu
from jax.experimental.pallas import tpu_sc as plsc

NB = 4
NW = 32
NCH = 28
SCAT = 128
SUB = 2048
NSUB = 7
CHUNK = SUB * NSUB
NPAD = CHUNK * NCH
SLABS = NPAD // 128
CSLABS = SLABS // NCH
N = 400000
PART = N % SUB

_MESH = plsc.VectorSubcoreMesh(core_axis_name="c", subcore_axis_name="s")
_CP = pltpu.CompilerParams(needs_layout_passes=False, use_tc_tiling_on_sc=False)


def _prep_body(c_ref, f_ref, planes_ref, b_ref, cnt_ref):
    i = pl.program_id(0)
    x = c_ref[:]
    f = f_ref[:]
    pc = jnp.concatenate([
        x[:, 0:1, :],
        (x[:, 3:4, :] + 0.5) * 0.05,
        (x[:, 2:3, :] + 0.5) * 0.05 - 40.0,
        (x[:, 1:2, :] + 0.5) * 0.1 - 3.0,
    ], axis=1)
    combined = jnp.concatenate([f, pc], axis=1)
    planes_ref[:] = combined.transpose(1, 0, 2)
    s_iota = lax.broadcasted_iota(jnp.int32, (CSLABS, 128), 0)
    l_iota = lax.broadcasted_iota(jnp.int32, (CSLABS, 128), 1)
    grow = 128 * (CSLABS * i + s_iota) + l_iota
    bi = jnp.where(grow < N, x[:, 0, :].astype(jnp.int32), NB)
    b_ref[:] = bi
    cl = lax.broadcasted_iota(jnp.int32, (1, 128), 1)
    cv = jnp.zeros((1, 128), jnp.int32)
    for b in range(NB):
        cv = jnp.where(cl == b, jnp.sum((bi == b).astype(jnp.int32)), cv)
    cnt_ref[:] = cv.reshape(1, 1, 128)


def _tc_prep(coords_slab, feats_slab):
    return pl.pallas_call(
        _prep_body,
        grid=(NCH,),
        in_specs=[
            pl.BlockSpec((CSLABS, 4, 128), lambda i: (i, 0, 0)),
            pl.BlockSpec((CSLABS, 4, 128), lambda i: (i, 0, 0)),
        ],
        out_specs=[
            pl.BlockSpec((8, CSLABS, 128), lambda i: (0, i, 0)),
            pl.BlockSpec((CSLABS, 128), lambda i: (i, 0)),
            pl.BlockSpec((1, 1, 128), lambda i: (i, 0, 0)),
        ],
        out_shape=[
            jax.ShapeDtypeStruct((8, SLABS, 128), jnp.float32),
            jax.ShapeDtypeStruct((SLABS, 128), jnp.int32),
            jax.ShapeDtypeStruct((NCH, 1, 128), jnp.int32),
        ],
    )(coords_slab, feats_slab)


@functools.partial(
    pl.kernel,
    mesh=_MESH,
    compiler_params=_CP,
    out_type=jax.ShapeDtypeStruct((NPAD, 8), jnp.float32),
    scratch_types=[
        pltpu.VMEM((NCH, 1, 128), jnp.int32),
        pltpu.VMEM((2, SUB), jnp.int32),
        pltpu.VMEM((2, 8, SUB), jnp.float32),
        pltpu.VMEM((SUB, 8), jnp.float32),
        pltpu.VMEM((SUB // SCAT, SCAT), jnp.int32),
        pltpu.SemaphoreType.DMA,
        pltpu.SemaphoreType.DMA,
    ],
)
def _sc_scatter(planes_hbm, bidx_hbm, cnt_hbm, out_hbm,
                cntbuf, bbuf, pjbuf, pbuf, destbuf, sem, seml):
    wid = lax.axis_index("s") * 2 + lax.axis_index("c")
    base = wid * CHUNK
    lanes = lax.iota(jnp.int32, 16)
    zeros = jnp.zeros((16,), jnp.int32)

    pltpu.sync_copy(cnt_hbm, cntbuf)
    wid_v = jnp.full((16,), wid, jnp.int32)
    totv = zeros
    minev = zeros
    for w in range(NCH):
        c_w = cntbuf[w, 0, pl.ds(0, 16)]
        totv = totv + c_w
        minev = minev + jnp.where(jnp.full((16,), w, jnp.int32) < wid_v,
                                  c_w, zeros)
    startsv = plsc.cumsum(totv) - totv + minev
    runs0 = tuple(
        jnp.full((16,), jnp.sum(jnp.where(lanes == b, startsv, zeros)),
                 jnp.int32)
        for b in range(NB))

    def fire_loads(j, buf):
        row0 = base + j * SUB
        pltpu.async_copy(bidx_hbm.at[pl.ds(row0, SUB)], bbuf.at[buf], seml)
        pltpu.async_copy(planes_hbm.at[:, pl.ds(row0, SUB)], pjbuf.at[buf],
                         seml)

    def sub_body(j, runs):
        row0 = base + j * SUB
        buf = lax.rem(j, 2)
        pltpu.make_async_copy(bidx_hbm.at[pl.ds(base, SUB)],
                              bbuf.at[buf], seml).wait()
        pltpu.make_async_copy(planes_hbm.at[:, pl.ds(base, SUB)],
                              pjbuf.at[buf], seml).wait()

        @pl.when(j + 1 < NSUB)
        def _():
            fire_loads(j + 1, 1 - buf)

        handles = []
        for jj in range(SUB // SCAT):
            def dg(g, runs):
                off = jj * SCAT + g * 16
                rows = off + lanes
                for p in range(8):
                    v = pjbuf[buf, p, pl.ds(off, 16)]
                    plsc.store_scatter(pbuf, [rows, zeros + p], v)
                bvec = bbuf[buf, pl.ds(off, 16)]
                dest = zeros
                new_runs = []
                for b in range(NB):
                    m = bvec == b
                    mi = m.astype(jnp.int32)
                    excl = plsc.cumsum(mi) - mi
                    dest = dest + jnp.where(m, runs[b] + excl, 0)
                    new_runs.append(
                        runs[b] + plsc.all_reduce_population_count(m))
                dest = jnp.where(bvec >= NB, row0 + rows, dest)
                destbuf[jj, pl.ds(g * 16, 16)] = dest
                return tuple(new_runs)

            runs = lax.fori_loop(0, SCAT // 16, dg, runs)
            handles.append(pltpu.async_copy(
                pbuf.at[pl.ds(jj * SCAT, SCAT)],
                out_hbm.at[destbuf.at[jj]], sem))
        for h in handles:
            h.wait()
        return runs

    @pl.when(wid < NCH)
    def _():
        fire_loads(0, 0)
        lax.fori_loop(0, NSUB, sub_body, runs0)


@functools.partial(
    pl.kernel,
    mesh=_MESH,
    compiler_params=_CP,
    out_type=[
        jax.ShapeDtypeStruct((4, N), jnp.float32),
        jax.ShapeDtypeStruct((4, N), jnp.float32),
    ],
    scratch_types=[
        pltpu.VMEM((2, SUB, 8), jnp.float32),
        pltpu.VMEM((8, SUB), jnp.float32),
        pltpu.SemaphoreType.DMA,
        pltpu.SemaphoreType.DMA,
    ],
)
def _sc_unpack(packed_hbm, outf_hbm, outpc_hbm, qbuf, pjbuf, sem, seml):
    wid = lax.axis_index("s") * 2 + lax.axis_index("c")
    base = wid * CHUNK
    lanes = lax.iota(jnp.int32, 16)
    zeros = jnp.zeros((16,), jnp.int32)

    def fire_load(j, buf):
        pltpu.async_copy(packed_hbm.at[pl.ds(base + j * SUB, SUB)],
                         qbuf.at[buf], seml)

    def sub_body(j, carry):
        row0 = base + j * SUB
        buf = lax.rem(j, 2)

        @pl.when(row0 < N)
        def _():
            pltpu.make_async_copy(packed_hbm.at[pl.ds(base, SUB)],
                                  qbuf.at[buf], seml).wait()

            @pl.when(jnp.logical_and(j + 1 < NSUB, row0 + SUB < N))
            def _():
                fire_load(j + 1, 1 - buf)

            def grp(g, carry):
                off = g * 16
                rows = off + lanes
                for p in range(8):
                    w = plsc.load_gather(qbuf, [zeros + buf, rows, zeros + p])
                    pjbuf[p, pl.ds(off, 16)] = w
                return carry

            lax.fori_loop(0, SUB // 16, grp, 0)

            @pl.when(row0 + SUB <= N)
            def _():
                h1 = pltpu.async_copy(
                    pjbuf.at[pl.ds(0, 4)],
                    outf_hbm.at[:, pl.ds(row0, SUB)], sem)
                h2 = pltpu.async_copy(
                    pjbuf.at[pl.ds(4, 4)],
                    outpc_hbm.at[:, pl.ds(row0, SUB)], sem)
                h1.wait()
                h2.wait()

            @pl.when(row0 + SUB > N)
            def _():
                h1 = pltpu.async_copy(
                    pjbuf.at[pl.ds(0, 4), pl.ds(0, PART)],
                    outf_hbm.at[:, pl.ds(row0, PART)], sem)
                h2 = pltpu.async_copy(
                    pjbuf.at[pl.ds(4, 4), pl.ds(0, PART)],
                    outpc_hbm.at[:, pl.ds(row0, PART)], sem)
                h1.wait()
                h2.wait()

        return carry

    @pl.when(wid < NCH)
    def _():
        fire_load(0, 0)
        lax.fori_loop(0, NSUB, sub_body, 0)


def _slab(a):
    return jnp.swapaxes(jnp.swapaxes(a, 0, 1).reshape(4, N // 128, 128), 0, 1)


def kernel(voxel_features, voxel_coords, batch_size):
    planes3d, bidx2d, cnts = _tc_prep(_slab(voxel_coords),
                                      _slab(voxel_features))
    packed_sorted = _sc_scatter(planes3d.reshape(8, NPAD),
                                bidx2d.reshape(NPAD), cnts)
    outf2d, outpc2d = _sc_unpack(packed_sorted)
    return jnp.swapaxes(outf2d, 0, 1), jnp.swapaxes(outpc2d, 0, 1)


# speedup vs baseline: 5.0945x; 1.0066x over previous
import functools

import jax
import jax.numpy as jnp
from jax import lax
from jax.experimental import pallas as pl
from jax.experimental.pallas import tpu as pltpu
from jax.experimental.pallas import tpu_sc as plsc

NB = 4
NW = 32
NCH = 28
SCAT = 128
SUB = 2048
NSUB = 7
CHUNK = SUB * NSUB
NPAD = CHUNK * NCH
SLABS = NPAD // 128
CSLABS = SLABS // NCH
N = 400000
PART = N % SUB

_MESH = plsc.VectorSubcoreMesh(core_axis_name="c", subcore_axis_name="s")
_CP = pltpu.CompilerParams(needs_layout_passes=False, use_tc_tiling_on_sc=False)


def _prep_body(c_ref, f_ref, planes_ref, b_ref, cnt_ref):
    i = pl.program_id(0)
    x = c_ref[:]
    f = f_ref[:]
    for p in range(4):
        planes_ref[p] = f[:, p, :]
    planes_ref[4] = x[:, 0, :]
    planes_ref[5] = (x[:, 3, :] + 0.5) * 0.05
    planes_ref[6] = (x[:, 2, :] + 0.5) * 0.05 - 40.0
    planes_ref[7] = (x[:, 1, :] + 0.5) * 0.1 - 3.0
    s_iota = lax.broadcasted_iota(jnp.int32, (CSLABS, 128), 0)
    l_iota = lax.broadcasted_iota(jnp.int32, (CSLABS, 128), 1)
    grow = 128 * (CSLABS * i + s_iota) + l_iota
    bi = jnp.where(grow < N, x[:, 0, :].astype(jnp.int32), NB)
    b_ref[:] = bi
    cl = lax.broadcasted_iota(jnp.int32, (1, 128), 1)
    cv = jnp.zeros((1, 128), jnp.int32)
    for b in range(NB):
        cv = jnp.where(cl == b, jnp.sum((bi == b).astype(jnp.int32)), cv)
    cnt_ref[:] = cv.reshape(1, 1, 128)


def _tc_prep(coords_slab, feats_slab):
    return pl.pallas_call(
        _prep_body,
        grid=(NCH,),
        in_specs=[
            pl.BlockSpec((CSLABS, 4, 128), lambda i: (i, 0, 0)),
            pl.BlockSpec((CSLABS, 4, 128), lambda i: (i, 0, 0)),
        ],
        out_specs=[
            pl.BlockSpec((8, CSLABS, 128), lambda i: (0, i, 0)),
            pl.BlockSpec((CSLABS, 128), lambda i: (i, 0)),
            pl.BlockSpec((1, 1, 128), lambda i: (i, 0, 0)),
        ],
        out_shape=[
            jax.ShapeDtypeStruct((8, SLABS, 128), jnp.float32),
            jax.ShapeDtypeStruct((SLABS, 128), jnp.int32),
            jax.ShapeDtypeStruct((NCH, 1, 128), jnp.int32),
        ],
    )(coords_slab, feats_slab)


@functools.partial(
    pl.kernel,
    mesh=_MESH,
    compiler_params=_CP,
    out_type=jax.ShapeDtypeStruct((NPAD, 8), jnp.float32),
    scratch_types=[
        pltpu.VMEM((NCH, 1, 128), jnp.int32),
        pltpu.VMEM((2, SUB), jnp.int32),
        pltpu.VMEM((2, 8, SUB), jnp.float32),
        pltpu.VMEM((2, SUB, 8), jnp.float32),
        pltpu.VMEM((2, SUB // SCAT, SCAT), jnp.int32),
        pltpu.SemaphoreType.DMA,
        pltpu.SemaphoreType.DMA,
    ],
)
def _sc_scatter(planes_hbm, bidx_hbm, cnt_hbm, out_hbm,
                cntbuf, bbuf, pjbuf, pbuf, destbuf, sem, seml):
    wid = lax.axis_index("s") * 2 + lax.axis_index("c")
    base = wid * CHUNK
    lanes = lax.iota(jnp.int32, 16)
    zeros = jnp.zeros((16,), jnp.int32)

    pltpu.sync_copy(cnt_hbm, cntbuf)
    wid_v = jnp.full((16,), wid, jnp.int32)
    totv = zeros
    minev = zeros
    for w in range(NCH):
        c_w = cntbuf[w, 0, pl.ds(0, 16)]
        totv = totv + c_w
        minev = minev + jnp.where(jnp.full((16,), w, jnp.int32) < wid_v,
                                  c_w, zeros)
    startsv = plsc.cumsum(totv) - totv + minev
    runs0 = tuple(
        jnp.full((16,), jnp.sum(jnp.where(lanes == b, startsv, zeros)),
                 jnp.int32)
        for b in range(NB))

    def fire_loads(j, buf):
        row0 = base + j * SUB
        pltpu.async_copy(bidx_hbm.at[pl.ds(row0, SUB)], bbuf.at[buf], seml)
        pltpu.async_copy(planes_hbm.at[:, pl.ds(row0, SUB)], pjbuf.at[buf],
                         seml)

    def sub_body(j, runs):
        row0 = base + j * SUB
        buf = lax.rem(j, 2)
        pltpu.make_async_copy(bidx_hbm.at[pl.ds(base, SUB)],
                              bbuf.at[buf], seml).wait()
        pltpu.make_async_copy(planes_hbm.at[:, pl.ds(base, SUB)],
                              pjbuf.at[buf], seml).wait()

        @pl.when(j + 1 < NSUB)
        def _():
            fire_loads(j + 1, 1 - buf)

        @pl.when(j >= 2)
        def _():
            pltpu.make_async_copy(out_hbm.at[pl.ds(base, SUB)],
                                  pbuf.at[buf], sem).wait()

        for jj in range(SUB // SCAT):
            def dg(g, runs):
                off = jj * SCAT + g * 16
                rows = off + lanes
                for p in range(8):
                    v = pjbuf[buf, p, pl.ds(off, 16)]
                    plsc.store_scatter(pbuf, [zeros + buf, rows, zeros + p], v)
                bvec = bbuf[buf, pl.ds(off, 16)]
                dest = zeros
                new_runs = []
                for b in range(NB):
                    m = bvec == b
                    mi = m.astype(jnp.int32)
                    excl = plsc.cumsum(mi) - mi
                    dest = dest + jnp.where(m, runs[b] + excl, 0)
                    new_runs.append(
                        runs[b] + plsc.all_reduce_population_count(m))
                dest = jnp.where(bvec >= NB, row0 + rows, dest)
                destbuf[buf, jj, pl.ds(g * 16, 16)] = dest
                return tuple(new_runs)

            runs = lax.fori_loop(0, SCAT // 16, dg, runs)
            pltpu.async_copy(
                pbuf.at[buf, pl.ds(jj * SCAT, SCAT)],
                out_hbm.at[destbuf.at[buf, jj]], sem)
        return runs

    @pl.when(wid < NCH)
    def _():
        fire_loads(0, 0)
        lax.fori_loop(0, NSUB, sub_body, runs0)
        pltpu.make_async_copy(out_hbm.at[pl.ds(base, SUB)],
                              pbuf.at[0], sem).wait()
        pltpu.make_async_copy(out_hbm.at[pl.ds(base, SUB)],
                              pbuf.at[1], sem).wait()


@functools.partial(
    pl.kernel,
    mesh=_MESH,
    compiler_params=_CP,
    out_type=[
        jax.ShapeDtypeStruct((4, N), jnp.float32),
        jax.ShapeDtypeStruct((4, N), jnp.float32),
    ],
    scratch_types=[
        pltpu.VMEM((2, SUB, 8), jnp.float32),
        pltpu.VMEM((8, SUB), jnp.float32),
        pltpu.SemaphoreType.DMA,
        pltpu.SemaphoreType.DMA,
    ],
)
def _sc_unpack(packed_hbm, outf_hbm, outpc_hbm, qbuf, pjbuf, sem, seml):
    wid = lax.axis_index("s") * 2 + lax.axis_index("c")
    base = wid * CHUNK
    lanes = lax.iota(jnp.int32, 16)
    zeros = jnp.zeros((16,), jnp.int32)

    def fire_load(j, buf):
        pltpu.async_copy(packed_hbm.at[pl.ds(base + j * SUB, SUB)],
                         qbuf.at[buf], seml)

    def sub_body(j, carry):
        row0 = base + j * SUB
        buf = lax.rem(j, 2)

        @pl.when(row0 < N)
        def _():
            pltpu.make_async_copy(packed_hbm.at[pl.ds(base, SUB)],
                                  qbuf.at[buf], seml).wait()

            @pl.when(jnp.logical_and(j + 1 < NSUB, row0 + SUB < N))
            def _():
                fire_load(j + 1, 1 - buf)

            def grp(g, carry):
                off = g * 16
                rows = off + lanes
                for p in range(8):
                    w = plsc.load_gather(qbuf, [zeros + buf, rows, zeros + p])
                    pjbuf[p, pl.ds(off, 16)] = w
                return carry

            lax.fori_loop(0, SUB // 16, grp, 0)

            @pl.when(row0 + SUB <= N)
            def _():
                h1 = pltpu.async_copy(
                    pjbuf.at[pl.ds(0, 4)],
                    outf_hbm.at[:, pl.ds(row0, SUB)], sem)
                h2 = pltpu.async_copy(
                    pjbuf.at[pl.ds(4, 4)],
                    outpc_hbm.at[:, pl.ds(row0, SUB)], sem)
                h1.wait()
                h2.wait()

            @pl.when(row0 + SUB > N)
            def _():
                h1 = pltpu.async_copy(
                    pjbuf.at[pl.ds(0, 4), pl.ds(0, PART)],
                    outf_hbm.at[:, pl.ds(row0, PART)], sem)
                h2 = pltpu.async_copy(
                    pjbuf.at[pl.ds(4, 4), pl.ds(0, PART)],
                    outpc_hbm.at[:, pl.ds(row0, PART)], sem)
                h1.wait()
                h2.wait()

        return carry

    @pl.when(wid < NCH)
    def _():
        fire_load(0, 0)
        lax.fori_loop(0, NSUB, sub_body, 0)


def _slab(a):
    return jnp.swapaxes(jnp.swapaxes(a, 0, 1).reshape(4, N // 128, 128), 0, 1)


def kernel(voxel_features, voxel_coords, batch_size):
    planes3d, bidx2d, cnts = _tc_prep(_slab(voxel_coords),
                                      _slab(voxel_features))
    packed_sorted = _sc_scatter(planes3d.reshape(8, NPAD),
                                bidx2d.reshape(NPAD), cnts)
    outf2d, outpc2d = _sc_unpack(packed_sorted)
    return jnp.swapaxes(outf2d, 0, 1), jnp.swapaxes(outpc2d, 0, 1)
